# baseline (speedup 1.0000x reference)
import functools

import jax
import jax.numpy as jnp
from jax import lax
from jax.experimental import pallas as pl
from jax.experimental.pallas import tpu as pltpu
from jax.experimental.pallas import tpu_sc as plsc

EMBED = 32
VOCAB = 1000000
SEQ = 20
HALF = 16

NUM_WORKERS = 32
TOKENS = 4096 * 26
TOK_PER_W = TOKENS // NUM_WORKERS
CHUNK_TOK = 64
CHUNKS = TOK_PER_W // CHUNK_TOK
PAIRS = CHUNKS // 2
ROWS_PER_CHUNK = CHUNK_TOK * SEQ
IDX_PER_GATHER = 128
GATHERS = ROWS_PER_CHUNK // IDX_PER_GATHER


B_DIM = 4096
N_DIM = 26
B_PER_W = B_DIM // NUM_WORKERS
GB = 16
GROUPS = B_PER_W // GB
PLANE = N_DIM * SEQ
FLAT_PER_G = GB * PLANE
FLAT_PER_W = B_PER_W * PLANE


def _compact_body(seq, outf, stage_v, comp_v):
    wid = lax.axis_index("s") * 2 + lax.axis_index("c")
    b0w = wid * B_PER_W
    lanes = lax.iota(jnp.int32, 16)

    def group_body(g, carry):
        pltpu.sync_copy(seq.at[pl.ds(b0w + g * GB, GB)], stage_v)
        for p in range(GB):
            for n in range(N_DIM):
                pos = p * PLANE + n * SEQ
                comp_v[pl.ds(pos, 16)] = stage_v[p, n, pl.ds(0, 16)]
                comp_v[pl.ds(pos + 4, 16)] = stage_v[p, n, pl.ds(4, 16)]
        pltpu.sync_copy(
            comp_v.at[pl.ds(0, FLAT_PER_G)],
            outf.at[pl.ds(wid * FLAT_PER_W + g * FLAT_PER_G, FLAT_PER_G)],
        )
        return carry

    lax.fori_loop(0, GROUPS, group_body, 0)


def _fire(table, idx, idx_v, rows_v, sem, wid, c):
    i0 = (wid * TOK_PER_W + c * CHUNK_TOK) * SEQ
    pltpu.sync_copy(idx.at[pl.ds(i0, ROWS_PER_CHUNK)], idx_v)
    for j in range(GATHERS):
        pltpu.async_copy(
            table.at[idx_v.at[pl.ds(j * IDX_PER_GATHER, IDX_PER_GATHER)]],
            rows_v.at[pl.ds(j * IDX_PER_GATHER, IDX_PER_GATHER)],
            sem,
        )


def _drain(table, idx_v, rows_v, sem):
    for j in range(GATHERS):
        pltpu.make_async_copy(
            table.at[idx_v.at[pl.ds(j * IDX_PER_GATHER, IDX_PER_GATHER)]],
            rows_v.at[pl.ds(j * IDX_PER_GATHER, IDX_PER_GATHER)],
            sem,
        ).wait()


def _compute(f_v, rows_v, out_v, out, wid, c):
    def tok_body(t, tcarry):
        base = t * SEQ
        acc0 = jnp.zeros((HALF,), jnp.float32)
        acc1 = jnp.zeros((HALF,), jnp.float32)
        for l in range(SEQ):
            acc0 = acc0 + f_v[l, pl.ds(0, HALF)] * rows_v[base + l, pl.ds(0, HALF)]
            acc1 = acc1 + f_v[l, pl.ds(HALF, HALF)] * rows_v[base + l, pl.ds(HALF, HALF)]
        out_v[t, pl.ds(0, HALF)] = acc0
        out_v[t, pl.ds(HALF, HALF)] = acc1
        return tcarry

    lax.fori_loop(0, CHUNK_TOK, tok_body, 0)
    tok0 = wid * TOK_PER_W + c * CHUNK_TOK
    pltpu.sync_copy(out_v, out.at[pl.ds(tok0, CHUNK_TOK)])


T_UNIT = 32
T_UNITS = VOCAB // T_UNIT
T_CHUNK_U = 32
T_ROWS_PER_CHUNK = T_CHUNK_U * T_UNIT


def _depad_body(table, out, buf_v, tsem0, tsem1):
    wid = lax.axis_index("s") * 2 + lax.axis_index("c")
    ucount = 976 + (wid < 18).astype(jnp.int32)
    ustart = wid * 976 + jnp.minimum(wid, 18)
    nchunks = 31
    last_u0 = ustart + ucount - T_CHUNK_U

    sems = (tsem0, tsem1)

    def chunk_rows(c):
        u0 = jnp.minimum(ustart + c * T_CHUNK_U, last_u0)
        return pl.multiple_of(u0 * T_UNIT, T_UNIT)

    def fire(c, b):
        pltpu.async_copy(
            table.at[pl.ds(chunk_rows(c), T_ROWS_PER_CHUNK)], buf_v.at[b], sems[b]
        )

    def drain(c, b):
        pltpu.make_async_copy(
            table.at[pl.ds(chunk_rows(c), T_ROWS_PER_CHUNK)], buf_v.at[b], sems[b]
        ).wait()

    def emit(c, b):
        pltpu.sync_copy(
            buf_v.at[b].reshape(T_ROWS_PER_CHUNK // 4, 128),
            out.at[pl.ds(pl.multiple_of(chunk_rows(c) // 4, T_UNIT // 4), T_ROWS_PER_CHUNK // 4)],
        )

    fire(0, 0)

    def pair_body(g, carry):
        c0 = g * 2
        fire(c0 + 1, 1)
        drain(c0, 0)
        emit(c0, 0)
        fire(c0 + 2, 0)
        drain(c0 + 1, 1)
        emit(c0 + 1, 1)
        return carry

    lax.fori_loop(0, (nchunks - 1) // 2, pair_body, 0)
    drain(nchunks - 1, 0)
    emit(nchunks - 1, 0)


def _gather_body(table, idx, f, out, idx_v, rows_v, f_v, out_v, sem0, sem1):
    wid = lax.axis_index("s") * 2 + lax.axis_index("c")
    pltpu.sync_copy(f, f_v)

    _fire(table, idx, idx_v.at[0], rows_v.at[0], sem0, wid, 0)

    def pair_body(g, carry):
        c0 = g * 2
        _fire(table, idx, idx_v.at[1], rows_v.at[1], sem1, wid, c0 + 1)
        _drain(table, idx_v.at[0], rows_v.at[0], sem0)
        _compute(f_v, rows_v.at[0], out_v, out, wid, c0)

        @pl.when(g < PAIRS - 1)
        def _():
            _fire(table, idx, idx_v.at[0], rows_v.at[0], sem0, wid, c0 + 2)

        _drain(table, idx_v.at[1], rows_v.at[1], sem1)
        _compute(f_v, rows_v.at[1], out_v, out, wid, c0 + 1)
        return carry

    lax.fori_loop(0, PAIRS, pair_body, 0)


@jax.jit
def kernel(input_sequence, embedding_table, f):
    B, N, L = input_sequence.shape
    mesh = plsc.VectorSubcoreMesh(core_axis_name="c", subcore_axis_name="s")
    idx1d = pl.kernel(
        _compact_body,
        out_type=jax.ShapeDtypeStruct((TOKENS * SEQ,), jnp.int32),
        mesh=mesh,
        scratch_types=[
            pltpu.VMEM((GB, N_DIM, SEQ), jnp.int32),
            pltpu.VMEM((FLAT_PER_G + 16,), jnp.int32),
        ],
    )(input_sequence)
    table_lin = lax.optimization_barrier(
        embedding_table.reshape(VOCAB * EMBED // 128, 128)
    )
    out = pl.kernel(
        _gather_body,
        out_type=jax.ShapeDtypeStruct((TOKENS, EMBED), jnp.float32),
        mesh=mesh,
        scratch_types=[
            pltpu.VMEM((2, ROWS_PER_CHUNK), jnp.int32),
            pltpu.VMEM((2, ROWS_PER_CHUNK, EMBED), jnp.float32),
            pltpu.VMEM((SEQ, EMBED), jnp.float32),
            pltpu.VMEM((CHUNK_TOK, EMBED), jnp.float32),
            pltpu.SemaphoreType.DMA,
            pltpu.SemaphoreType.DMA,
        ],
        compiler_params=pltpu.CompilerParams(use_tc_tiling_on_sc=False),
    )(table_lin.reshape(VOCAB, EMBED), idx1d, f)
    return out.reshape(B, N, EMBED)


# speedup vs baseline: 1.1269x; 1.1269x over previous
import functools

import jax
import jax.numpy as jnp
from jax import lax
from jax.experimental import pallas as pl
from jax.experimental.pallas import tpu as pltpu
from jax.experimental.pallas import tpu_sc as plsc

EMBED = 32
VOCAB = 1000000
SEQ = 20
HALF = 16

NUM_WORKERS = 32
TOKENS = 4096 * 26
TOK_PER_W = TOKENS // NUM_WORKERS
CHUNK_TOK = 64
CHUNKS = TOK_PER_W // CHUNK_TOK
PAIRS = CHUNKS // 2
ROWS_PER_CHUNK = CHUNK_TOK * SEQ
IDX_PER_GATHER = 128
GATHERS = ROWS_PER_CHUNK // IDX_PER_GATHER


B_DIM = 4096
N_DIM = 26
B_PER_W = B_DIM // NUM_WORKERS
GB = 16
GROUPS = B_PER_W // GB
PLANE = N_DIM * SEQ
FLAT_PER_G = GB * PLANE
FLAT_PER_W = B_PER_W * PLANE


def _compact_body(seq, outf, stage_v, comp_v):
    wid = lax.axis_index("s") * 2 + lax.axis_index("c")
    b0w = wid * B_PER_W
    lanes = lax.iota(jnp.int32, 16)

    def group_body(g, carry):
        pltpu.sync_copy(seq.at[pl.ds(b0w + g * GB, GB)], stage_v)
        for p in range(GB):
            for n in range(N_DIM):
                pos = p * PLANE + n * SEQ
                comp_v[pl.ds(pos, 16)] = stage_v[p, n, pl.ds(0, 16)]
                comp_v[pl.ds(pos + 4, 16)] = stage_v[p, n, pl.ds(4, 16)]
        pltpu.sync_copy(
            comp_v.at[pl.ds(0, FLAT_PER_G)],
            outf.at[pl.ds(wid * FLAT_PER_W + g * FLAT_PER_G, FLAT_PER_G)],
        )
        return carry

    lax.fori_loop(0, GROUPS, group_body, 0)


def _fire(table, idx, idx_v, rows_v, sem, wid, c):
    i0 = (wid * TOK_PER_W + c * CHUNK_TOK) * SEQ
    pltpu.sync_copy(idx.at[pl.ds(i0, ROWS_PER_CHUNK)], idx_v)
    for j in range(GATHERS):
        pltpu.async_copy(
            table.at[idx_v.at[pl.ds(j * IDX_PER_GATHER, IDX_PER_GATHER)]],
            rows_v.at[pl.ds(j * IDX_PER_GATHER, IDX_PER_GATHER)],
            sem,
        )


def _drain(table, idx_v, rows_v, sem):
    for j in range(GATHERS):
        pltpu.make_async_copy(
            table.at[idx_v.at[pl.ds(j * IDX_PER_GATHER, IDX_PER_GATHER)]],
            rows_v.at[pl.ds(j * IDX_PER_GATHER, IDX_PER_GATHER)],
            sem,
        ).wait()


def _compute(f_v, rows_v, out_v, out, wid, c):
    def tok_body(t, tcarry):
        base = t * SEQ
        acc0 = jnp.zeros((HALF,), jnp.float32)
        acc1 = jnp.zeros((HALF,), jnp.float32)
        for l in range(SEQ):
            acc0 = acc0 + f_v[l, pl.ds(0, HALF)] * rows_v[base + l, pl.ds(0, HALF)]
            acc1 = acc1 + f_v[l, pl.ds(HALF, HALF)] * rows_v[base + l, pl.ds(HALF, HALF)]
        out_v[t, pl.ds(0, HALF)] = acc0
        out_v[t, pl.ds(HALF, HALF)] = acc1
        return tcarry

    lax.fori_loop(0, CHUNK_TOK, tok_body, 0)
    tok0 = wid * TOK_PER_W + c * CHUNK_TOK
    pltpu.sync_copy(out_v, out.at[pl.ds(tok0, CHUNK_TOK)])


TVB = 4096
TGRID = (VOCAB + TVB - 1) // TVB


def _transpose_body(x_ref, o_ref):
    x = x_ref[...]
    y = x.T.reshape(TVB // 4, 4, EMBED)
    for q in range(4):
        o_ref[:, pl.ds(q * EMBED, EMBED)] = y[:, q, :]


def _gather_body(table, idx, f, out, idx_v, rows_v, f_v, out_v, sem0, sem1):
    wid = lax.axis_index("s") * 2 + lax.axis_index("c")
    pltpu.sync_copy(f, f_v)

    _fire(table, idx, idx_v.at[0], rows_v.at[0], sem0, wid, 0)

    def pair_body(g, carry):
        c0 = g * 2
        _fire(table, idx, idx_v.at[1], rows_v.at[1], sem1, wid, c0 + 1)
        _drain(table, idx_v.at[0], rows_v.at[0], sem0)
        _compute(f_v, rows_v.at[0], out_v, out, wid, c0)

        @pl.when(g < PAIRS - 1)
        def _():
            _fire(table, idx, idx_v.at[0], rows_v.at[0], sem0, wid, c0 + 2)

        _drain(table, idx_v.at[1], rows_v.at[1], sem1)
        _compute(f_v, rows_v.at[1], out_v, out, wid, c0 + 1)
        return carry

    lax.fori_loop(0, PAIRS, pair_body, 0)


@jax.jit
def kernel(input_sequence, embedding_table, f):
    B, N, L = input_sequence.shape
    mesh = plsc.VectorSubcoreMesh(core_axis_name="c", subcore_axis_name="s")
    idx1d = pl.kernel(
        _compact_body,
        out_type=jax.ShapeDtypeStruct((TOKENS * SEQ,), jnp.int32),
        mesh=mesh,
        scratch_types=[
            pltpu.VMEM((GB, N_DIM, SEQ), jnp.int32),
            pltpu.VMEM((FLAT_PER_G + 16,), jnp.int32),
        ],
    )(input_sequence)
    table128 = pl.pallas_call(
        _transpose_body,
        grid=(TGRID,),
        in_specs=[pl.BlockSpec((EMBED, TVB), lambda i: (0, i))],
        out_specs=pl.BlockSpec((TVB * EMBED // 128, 128), lambda i: (i, 0)),
        out_shape=jax.ShapeDtypeStruct((VOCAB * EMBED // 128, 128), jnp.float32),
    )(embedding_table.T)
    table_lin = table128
    out = pl.kernel(
        _gather_body,
        out_type=jax.ShapeDtypeStruct((TOKENS, EMBED), jnp.float32),
        mesh=mesh,
        scratch_types=[
            pltpu.VMEM((2, ROWS_PER_CHUNK), jnp.int32),
            pltpu.VMEM((2, ROWS_PER_CHUNK, EMBED), jnp.float32),
            pltpu.VMEM((SEQ, EMBED), jnp.float32),
            pltpu.VMEM((CHUNK_TOK, EMBED), jnp.float32),
            pltpu.SemaphoreType.DMA,
            pltpu.SemaphoreType.DMA,
        ],
        compiler_params=pltpu.CompilerParams(use_tc_tiling_on_sc=False),
    )(table_lin.reshape(VOCAB, EMBED), idx1d, f)
    return out.reshape(B, N, EMBED)


# speedup vs baseline: 1.1831x; 1.0499x over previous
import functools

import jax
import jax.numpy as jnp
from jax import lax
from jax.experimental import pallas as pl
from jax.experimental.pallas import tpu as pltpu
from jax.experimental.pallas import tpu_sc as plsc

EMBED = 32
VOCAB = 1000000
SEQ = 20
HALF = 16

NUM_WORKERS = 32
TOKENS = 4096 * 26
TOK_PER_W = TOKENS // NUM_WORKERS
CHUNK_TOK = 64
CHUNKS = TOK_PER_W // CHUNK_TOK
PAIRS = CHUNKS // 2
ROWS_PER_CHUNK = CHUNK_TOK * SEQ
IDX_PER_GATHER = 128
GATHERS = ROWS_PER_CHUNK // IDX_PER_GATHER


B_DIM = 4096
N_DIM = 26
B_PER_W = B_DIM // NUM_WORKERS
GB = 16
GROUPS = B_PER_W // GB
PLANE = N_DIM * SEQ
FLAT_PER_G = GB * PLANE
FLAT_PER_W = B_PER_W * PLANE


def _compact_body(seq, outf, stage_v, comp_v):
    wid = lax.axis_index("s") * 2 + lax.axis_index("c")
    b0w = wid * B_PER_W
    lanes = lax.iota(jnp.int32, 16)

    def group_body(g, carry):
        pltpu.sync_copy(seq.at[pl.ds(b0w + g * GB, GB)], stage_v)
        for p in range(GB):
            for n in range(N_DIM):
                pos = p * PLANE + n * SEQ
                comp_v[pl.ds(pos, 16)] = stage_v[p, n, pl.ds(0, 16)]
                comp_v[pl.ds(pos + 4, 16)] = stage_v[p, n, pl.ds(4, 16)]
        pltpu.sync_copy(
            comp_v.at[pl.ds(0, FLAT_PER_G)],
            outf.at[pl.ds(wid * FLAT_PER_W + g * FLAT_PER_G, FLAT_PER_G)],
        )
        return carry

    lax.fori_loop(0, GROUPS, group_body, 0)


def _stage_idx(idx, idx_v, isem, wid, c):
    i0 = (wid * TOK_PER_W + c * CHUNK_TOK) * SEQ
    pltpu.async_copy(idx.at[pl.ds(i0, ROWS_PER_CHUNK)], idx_v, isem)


def _wait_idx(idx, idx_v, isem, wid, c):
    pltpu.make_async_copy(
        idx.at[pl.ds((wid * TOK_PER_W + c * CHUNK_TOK) * SEQ, ROWS_PER_CHUNK)],
        idx_v,
        isem,
    ).wait()


def _fire(table, idx_v, rows_v, sem):
    for j in range(GATHERS):
        pltpu.async_copy(
            table.at[idx_v.at[pl.ds(j * IDX_PER_GATHER, IDX_PER_GATHER)]],
            rows_v.at[pl.ds(j * IDX_PER_GATHER, IDX_PER_GATHER)],
            sem,
        )


def _drain(table, idx_v, rows_v, sem):
    for j in range(GATHERS):
        pltpu.make_async_copy(
            table.at[idx_v.at[pl.ds(j * IDX_PER_GATHER, IDX_PER_GATHER)]],
            rows_v.at[pl.ds(j * IDX_PER_GATHER, IDX_PER_GATHER)],
            sem,
        ).wait()


def _compute(f_v, rows_v, out_v, out, wid, c):
    def tok_body(t, tcarry):
        base = t * SEQ
        acc0 = jnp.zeros((HALF,), jnp.float32)
        acc1 = jnp.zeros((HALF,), jnp.float32)
        for l in range(SEQ):
            acc0 = acc0 + f_v[l, pl.ds(0, HALF)] * rows_v[base + l, pl.ds(0, HALF)]
            acc1 = acc1 + f_v[l, pl.ds(HALF, HALF)] * rows_v[base + l, pl.ds(HALF, HALF)]
        out_v[t, pl.ds(0, HALF)] = acc0
        out_v[t, pl.ds(HALF, HALF)] = acc1
        return tcarry

    lax.fori_loop(0, CHUNK_TOK, tok_body, 0)
    tok0 = wid * TOK_PER_W + c * CHUNK_TOK
    pltpu.sync_copy(out_v, out.at[pl.ds(tok0, CHUNK_TOK)])


TVB = 4096
TGRID = (VOCAB + TVB - 1) // TVB


def _transpose_body(x_ref, o_ref):
    x = x_ref[...]
    y = x.T.reshape(TVB // 4, 4, EMBED)
    for q in range(4):
        o_ref[:, pl.ds(q * EMBED, EMBED)] = y[:, q, :]


def _gather_body(
    table, idx, f, out, idx_v, rows_v, f_v, out_v, sem0, sem1, isem0, isem1
):
    wid = lax.axis_index("s") * 2 + lax.axis_index("c")
    pltpu.sync_copy(f, f_v)

    _stage_idx(idx, idx_v.at[0], isem0, wid, 0)
    _wait_idx(idx, idx_v.at[0], isem0, wid, 0)
    _fire(table, idx_v.at[0], rows_v.at[0], sem0)
    _stage_idx(idx, idx_v.at[1], isem1, wid, 1)

    def pair_body(g, carry):
        c0 = g * 2
        _wait_idx(idx, idx_v.at[1], isem1, wid, c0 + 1)
        _fire(table, idx_v.at[1], rows_v.at[1], sem1)
        _drain(table, idx_v.at[0], rows_v.at[0], sem0)

        @pl.when(g < PAIRS - 1)
        def _():
            _stage_idx(idx, idx_v.at[0], isem0, wid, c0 + 2)

        _compute(f_v, rows_v.at[0], out_v, out, wid, c0)

        @pl.when(g < PAIRS - 1)
        def _():
            _wait_idx(idx, idx_v.at[0], isem0, wid, c0 + 2)
            _fire(table, idx_v.at[0], rows_v.at[0], sem0)

        _drain(table, idx_v.at[1], rows_v.at[1], sem1)

        @pl.when(g < PAIRS - 1)
        def _():
            _stage_idx(idx, idx_v.at[1], isem1, wid, c0 + 3)

        _compute(f_v, rows_v.at[1], out_v, out, wid, c0 + 1)
        return carry

    lax.fori_loop(0, PAIRS, pair_body, 0)


@jax.jit
def kernel(input_sequence, embedding_table, f):
    B, N, L = input_sequence.shape
    mesh = plsc.VectorSubcoreMesh(core_axis_name="c", subcore_axis_name="s")
    idx1d = pl.kernel(
        _compact_body,
        out_type=jax.ShapeDtypeStruct((TOKENS * SEQ,), jnp.int32),
        mesh=mesh,
        scratch_types=[
            pltpu.VMEM((GB, N_DIM, SEQ), jnp.int32),
            pltpu.VMEM((FLAT_PER_G + 16,), jnp.int32),
        ],
    )(input_sequence)
    table128 = pl.pallas_call(
        _transpose_body,
        grid=(TGRID,),
        in_specs=[pl.BlockSpec((EMBED, TVB), lambda i: (0, i))],
        out_specs=pl.BlockSpec((TVB * EMBED // 128, 128), lambda i: (i, 0)),
        out_shape=jax.ShapeDtypeStruct((VOCAB * EMBED // 128, 128), jnp.float32),
    )(embedding_table.T)
    table_lin = table128
    out = pl.kernel(
        _gather_body,
        out_type=jax.ShapeDtypeStruct((TOKENS, EMBED), jnp.float32),
        mesh=mesh,
        scratch_types=[
            pltpu.VMEM((2, ROWS_PER_CHUNK), jnp.int32),
            pltpu.VMEM((2, ROWS_PER_CHUNK, EMBED), jnp.float32),
            pltpu.VMEM((SEQ, EMBED), jnp.float32),
            pltpu.VMEM((CHUNK_TOK, EMBED), jnp.float32),
            pltpu.SemaphoreType.DMA,
            pltpu.SemaphoreType.DMA,
            pltpu.SemaphoreType.DMA,
            pltpu.SemaphoreType.DMA,
        ],
        compiler_params=pltpu.CompilerParams(use_tc_tiling_on_sc=False),
    )(table_lin.reshape(VOCAB, EMBED), idx1d, f)
    return out.reshape(B, N, EMBED)


# speedup vs baseline: 1.2249x; 1.0353x over previous
import functools

import jax
import jax.numpy as jnp
from jax import lax
from jax.experimental import pallas as pl
from jax.experimental.pallas import tpu as pltpu
from jax.experimental.pallas import tpu_sc as plsc

EMBED = 32
VOCAB = 1000000
SEQ = 20
HALF = 16

NUM_WORKERS = 32
TOKENS = 4096 * 26
TOK_PER_W = TOKENS // NUM_WORKERS
CHUNK_TOK = 64
CHUNKS = TOK_PER_W // CHUNK_TOK
PAIRS = CHUNKS // 2
ROWS_PER_CHUNK = CHUNK_TOK * SEQ
IDX_PER_GATHER = 128
GATHERS = ROWS_PER_CHUNK // IDX_PER_GATHER


B_DIM = 4096
N_DIM = 26
B_PER_W = B_DIM // NUM_WORKERS
GB = 16
GROUPS = B_PER_W // GB
PLANE = N_DIM * SEQ
FLAT_PER_G = GB * PLANE
FLAT_PER_W = B_PER_W * PLANE


def _compact_body(seq, outf, stage_v, comp_v):
    wid = lax.axis_index("s") * 2 + lax.axis_index("c")
    b0w = wid * B_PER_W
    lanes = lax.iota(jnp.int32, 16)

    def group_body(g, carry):
        pltpu.sync_copy(seq.at[pl.ds(b0w + g * GB, GB)], stage_v)
        for p in range(GB):
            for n in range(N_DIM):
                pos = p * PLANE + n * SEQ
                comp_v[pl.ds(pos, 16)] = stage_v[p, n, pl.ds(0, 16)]
                comp_v[pl.ds(pos + 4, 16)] = stage_v[p, n, pl.ds(4, 16)]
        pltpu.sync_copy(
            comp_v.at[pl.ds(0, FLAT_PER_G)],
            outf.at[pl.ds(wid * FLAT_PER_W + g * FLAT_PER_G, FLAT_PER_G)],
        )
        return carry

    lax.fori_loop(0, GROUPS, group_body, 0)


def _stage_idx(idx, idx_v, isem, wid, c):
    i0 = (wid * TOK_PER_W + c * CHUNK_TOK) * SEQ
    pltpu.async_copy(idx.at[pl.ds(i0, ROWS_PER_CHUNK)], idx_v, isem)


def _wait_idx(idx, idx_v, isem, wid, c):
    pltpu.make_async_copy(
        idx.at[pl.ds((wid * TOK_PER_W + c * CHUNK_TOK) * SEQ, ROWS_PER_CHUNK)],
        idx_v,
        isem,
    ).wait()


def _fire(table, idx_v, rows_v, sem):
    for j in range(GATHERS):
        pltpu.async_copy(
            table.at[idx_v.at[pl.ds(j * IDX_PER_GATHER, IDX_PER_GATHER)]],
            rows_v.at[pl.ds(j * IDX_PER_GATHER, IDX_PER_GATHER)],
            sem,
        )


def _drain(table, idx_v, rows_v, sem):
    for j in range(GATHERS):
        pltpu.make_async_copy(
            table.at[idx_v.at[pl.ds(j * IDX_PER_GATHER, IDX_PER_GATHER)]],
            rows_v.at[pl.ds(j * IDX_PER_GATHER, IDX_PER_GATHER)],
            sem,
        ).wait()


def _compute(f_v, rows_v, out_v, out, wid, c):
    def tok_body(t, tcarry):
        base = t * SEQ
        acc0 = jnp.zeros((HALF,), jnp.float32)
        acc1 = jnp.zeros((HALF,), jnp.float32)
        for l in range(SEQ):
            acc0 = acc0 + f_v[l, pl.ds(0, HALF)] * rows_v[base + l, pl.ds(0, HALF)]
            acc1 = acc1 + f_v[l, pl.ds(HALF, HALF)] * rows_v[base + l, pl.ds(HALF, HALF)]
        out_v[t, pl.ds(0, HALF)] = acc0
        out_v[t, pl.ds(HALF, HALF)] = acc1
        return tcarry

    lax.fori_loop(0, CHUNK_TOK, tok_body, 0)
    tok0 = wid * TOK_PER_W + c * CHUNK_TOK
    pltpu.sync_copy(out_v, out.at[pl.ds(tok0, CHUNK_TOK)])


TVB = 8192
TGRID = (VOCAB + TVB - 1) // TVB


def _transpose_body(x_ref, o_ref):
    x = x_ref[...]
    y = x.T.reshape(TVB // 4, 4, EMBED)
    for q in range(4):
        o_ref[:, pl.ds(q * EMBED, EMBED)] = y[:, q, :]


def _gather_body(
    table, idx, f, out, idx_v, rows_v, f_v, out_v, sem0, sem1, isem0, isem1
):
    wid = lax.axis_index("s") * 2 + lax.axis_index("c")
    pltpu.sync_copy(f, f_v)

    _stage_idx(idx, idx_v.at[0], isem0, wid, 0)
    _wait_idx(idx, idx_v.at[0], isem0, wid, 0)
    _fire(table, idx_v.at[0], rows_v.at[0], sem0)
    _stage_idx(idx, idx_v.at[1], isem1, wid, 1)

    def pair_body(g, carry):
        c0 = g * 2
        _wait_idx(idx, idx_v.at[1], isem1, wid, c0 + 1)
        _fire(table, idx_v.at[1], rows_v.at[1], sem1)
        _drain(table, idx_v.at[0], rows_v.at[0], sem0)

        @pl.when(g < PAIRS - 1)
        def _():
            _stage_idx(idx, idx_v.at[0], isem0, wid, c0 + 2)

        _compute(f_v, rows_v.at[0], out_v, out, wid, c0)

        @pl.when(g < PAIRS - 1)
        def _():
            _wait_idx(idx, idx_v.at[0], isem0, wid, c0 + 2)
            _fire(table, idx_v.at[0], rows_v.at[0], sem0)

        _drain(table, idx_v.at[1], rows_v.at[1], sem1)

        @pl.when(g < PAIRS - 1)
        def _():
            _stage_idx(idx, idx_v.at[1], isem1, wid, c0 + 3)

        _compute(f_v, rows_v.at[1], out_v, out, wid, c0 + 1)
        return carry

    lax.fori_loop(0, PAIRS, pair_body, 0)


@jax.jit
def kernel(input_sequence, embedding_table, f):
    B, N, L = input_sequence.shape
    mesh = plsc.VectorSubcoreMesh(core_axis_name="c", subcore_axis_name="s")
    idx1d = pl.kernel(
        _compact_body,
        out_type=jax.ShapeDtypeStruct((TOKENS * SEQ,), jnp.int32),
        mesh=mesh,
        scratch_types=[
            pltpu.VMEM((GB, N_DIM, SEQ), jnp.int32),
            pltpu.VMEM((FLAT_PER_G + 16,), jnp.int32),
        ],
    )(input_sequence)
    table128 = pl.pallas_call(
        _transpose_body,
        grid=(TGRID,),
        in_specs=[pl.BlockSpec((EMBED, TVB), lambda i: (0, i))],
        out_specs=pl.BlockSpec((TVB * EMBED // 128, 128), lambda i: (i, 0)),
        out_shape=jax.ShapeDtypeStruct((VOCAB * EMBED // 128, 128), jnp.float32),
    )(embedding_table.T)
    table_lin = table128
    out = pl.kernel(
        _gather_body,
        out_type=jax.ShapeDtypeStruct((TOKENS, EMBED), jnp.float32),
        mesh=mesh,
        scratch_types=[
            pltpu.VMEM((2, ROWS_PER_CHUNK), jnp.int32),
            pltpu.VMEM((2, ROWS_PER_CHUNK, EMBED), jnp.float32),
            pltpu.VMEM((SEQ, EMBED), jnp.float32),
            pltpu.VMEM((CHUNK_TOK, EMBED), jnp.float32),
            pltpu.SemaphoreType.DMA,
            pltpu.SemaphoreType.DMA,
            pltpu.SemaphoreType.DMA,
            pltpu.SemaphoreType.DMA,
        ],
        compiler_params=pltpu.CompilerParams(use_tc_tiling_on_sc=False),
    )(table_lin.reshape(VOCAB, EMBED), idx1d, f)
    return out.reshape(B, N, EMBED)


# speedup vs baseline: 1.2450x; 1.0164x over previous
import functools

import jax
import jax.numpy as jnp
from jax import lax
from jax.experimental import pallas as pl
from jax.experimental.pallas import tpu as pltpu
from jax.experimental.pallas import tpu_sc as plsc

EMBED = 32
VOCAB = 1000000
SEQ = 20
HALF = 16

NUM_WORKERS = 32
TOKENS = 4096 * 26
TOK_PER_W = TOKENS // NUM_WORKERS
CHUNK_TOK = 64
CHUNKS = TOK_PER_W // CHUNK_TOK
PAIRS = CHUNKS // 2
ROWS_PER_CHUNK = CHUNK_TOK * SEQ
IDX_PER_GATHER = 128
GATHERS = ROWS_PER_CHUNK // IDX_PER_GATHER


B_DIM = 4096
N_DIM = 26
B_PER_W = B_DIM // NUM_WORKERS
GB = 16
GROUPS = B_PER_W // GB
PLANE = N_DIM * SEQ
FLAT_PER_G = GB * PLANE
FLAT_PER_W = B_PER_W * PLANE


def _compact_body(seq, outf, stage_v, comp_v):
    wid = lax.axis_index("s") * 2 + lax.axis_index("c")
    b0w = wid * B_PER_W
    lanes = lax.iota(jnp.int32, 16)

    def group_body(g, carry):
        pltpu.sync_copy(seq.at[pl.ds(b0w + g * GB, GB)], stage_v)
        for p in range(GB):
            for n in range(N_DIM):
                pos = p * PLANE + n * SEQ
                comp_v[pl.ds(pos, 16)] = stage_v[p, n, pl.ds(0, 16)]
                comp_v[pl.ds(pos + 4, 16)] = stage_v[p, n, pl.ds(4, 16)]
        pltpu.sync_copy(
            comp_v.at[pl.ds(0, FLAT_PER_G)],
            outf.at[pl.ds(wid * FLAT_PER_W + g * FLAT_PER_G, FLAT_PER_G)],
        )
        return carry

    lax.fori_loop(0, GROUPS, group_body, 0)


def _stage_idx(idx, idx_v, isem, wid, c):
    i0 = (wid * TOK_PER_W + c * CHUNK_TOK) * SEQ
    pltpu.async_copy(idx.at[pl.ds(i0, ROWS_PER_CHUNK)], idx_v, isem)


def _wait_idx(idx, idx_v, isem, wid, c):
    pltpu.make_async_copy(
        idx.at[pl.ds((wid * TOK_PER_W + c * CHUNK_TOK) * SEQ, ROWS_PER_CHUNK)],
        idx_v,
        isem,
    ).wait()


def _fire(table, idx_v, rows_v, sem):
    for j in range(GATHERS):
        pltpu.async_copy(
            table.at[idx_v.at[pl.ds(j * IDX_PER_GATHER, IDX_PER_GATHER)]],
            rows_v.at[pl.ds(j * IDX_PER_GATHER, IDX_PER_GATHER)],
            sem,
        )


def _drain(table, idx_v, rows_v, sem):
    for j in range(GATHERS):
        pltpu.make_async_copy(
            table.at[idx_v.at[pl.ds(j * IDX_PER_GATHER, IDX_PER_GATHER)]],
            rows_v.at[pl.ds(j * IDX_PER_GATHER, IDX_PER_GATHER)],
            sem,
        ).wait()


def _compute(f_v, rows_v, out_v, out, wid, c):
    def tok_body(t, tcarry):
        base = t * SEQ
        acc0 = jnp.zeros((HALF,), jnp.float32)
        acc1 = jnp.zeros((HALF,), jnp.float32)
        for l in range(SEQ):
            acc0 = acc0 + f_v[l, pl.ds(0, HALF)] * rows_v[base + l, pl.ds(0, HALF)]
            acc1 = acc1 + f_v[l, pl.ds(HALF, HALF)] * rows_v[base + l, pl.ds(HALF, HALF)]
        out_v[t, pl.ds(0, HALF)] = acc0
        out_v[t, pl.ds(HALF, HALF)] = acc1
        return tcarry

    lax.fori_loop(0, CHUNK_TOK, tok_body, 0)
    tok0 = wid * TOK_PER_W + c * CHUNK_TOK
    pltpu.sync_copy(out_v, out.at[pl.ds(tok0, CHUNK_TOK)])


TVB = 16384
TGRID = (VOCAB + TVB - 1) // TVB


def _transpose_body(x_ref, o_ref):
    x = x_ref[...]
    y = x.T.reshape(TVB // 4, 4, EMBED)
    for q in range(4):
        o_ref[:, pl.ds(q * EMBED, EMBED)] = y[:, q, :]


def _gather_body(
    table, idx, f, out, idx_v, rows_v, f_v, out_v, sem0, sem1, isem0, isem1
):
    wid = lax.axis_index("s") * 2 + lax.axis_index("c")
    pltpu.sync_copy(f, f_v)

    _stage_idx(idx, idx_v.at[0], isem0, wid, 0)
    _wait_idx(idx, idx_v.at[0], isem0, wid, 0)
    _fire(table, idx_v.at[0], rows_v.at[0], sem0)
    _stage_idx(idx, idx_v.at[1], isem1, wid, 1)

    def pair_body(g, carry):
        c0 = g * 2
        _wait_idx(idx, idx_v.at[1], isem1, wid, c0 + 1)
        _fire(table, idx_v.at[1], rows_v.at[1], sem1)
        _drain(table, idx_v.at[0], rows_v.at[0], sem0)

        @pl.when(g < PAIRS - 1)
        def _():
            _stage_idx(idx, idx_v.at[0], isem0, wid, c0 + 2)

        _compute(f_v, rows_v.at[0], out_v, out, wid, c0)

        @pl.when(g < PAIRS - 1)
        def _():
            _wait_idx(idx, idx_v.at[0], isem0, wid, c0 + 2)
            _fire(table, idx_v.at[0], rows_v.at[0], sem0)

        _drain(table, idx_v.at[1], rows_v.at[1], sem1)

        @pl.when(g < PAIRS - 1)
        def _():
            _stage_idx(idx, idx_v.at[1], isem1, wid, c0 + 3)

        _compute(f_v, rows_v.at[1], out_v, out, wid, c0 + 1)
        return carry

    lax.fori_loop(0, PAIRS, pair_body, 0)


@jax.jit
def kernel(input_sequence, embedding_table, f):
    B, N, L = input_sequence.shape
    mesh = plsc.VectorSubcoreMesh(core_axis_name="c", subcore_axis_name="s")
    idx1d = pl.kernel(
        _compact_body,
        out_type=jax.ShapeDtypeStruct((TOKENS * SEQ,), jnp.int32),
        mesh=mesh,
        scratch_types=[
            pltpu.VMEM((GB, N_DIM, SEQ), jnp.int32),
            pltpu.VMEM((FLAT_PER_G + 16,), jnp.int32),
        ],
    )(input_sequence)
    table128 = pl.pallas_call(
        _transpose_body,
        grid=(TGRID,),
        in_specs=[pl.BlockSpec((EMBED, TVB), lambda i: (0, i))],
        out_specs=pl.BlockSpec((TVB * EMBED // 128, 128), lambda i: (i, 0)),
        out_shape=jax.ShapeDtypeStruct((VOCAB * EMBED // 128, 128), jnp.float32),
    )(embedding_table.T)
    table_lin = table128
    out = pl.kernel(
        _gather_body,
        out_type=jax.ShapeDtypeStruct((TOKENS, EMBED), jnp.float32),
        mesh=mesh,
        scratch_types=[
            pltpu.VMEM((2, ROWS_PER_CHUNK), jnp.int32),
            pltpu.VMEM((2, ROWS_PER_CHUNK, EMBED), jnp.float32),
            pltpu.VMEM((SEQ, EMBED), jnp.float32),
            pltpu.VMEM((CHUNK_TOK, EMBED), jnp.float32),
            pltpu.SemaphoreType.DMA,
            pltpu.SemaphoreType.DMA,
            pltpu.SemaphoreType.DMA,
            pltpu.SemaphoreType.DMA,
        ],
        compiler_params=pltpu.CompilerParams(use_tc_tiling_on_sc=False),
    )(table_lin.reshape(VOCAB, EMBED), idx1d, f)
    return out.reshape(B, N, EMBED)
